# baseline (speedup 1.0000x reference)
import jax
import jax.numpy as jnp
from jax import lax
from jax.experimental import pallas as pl
from jax.experimental.pallas import tpu as pltpu
from jax.experimental.pallas import tpu_sc as plsc

_NC = 2
_NS = 16
_NW = _NC * _NS
_K = 128
_G = 8
_PREC = lax.Precision.HIGHEST


def _sc_segsum(tables, srcp, dstp, n_acc, ea=None):
    ngroups = srcp.shape[1]
    chunks = ngroups * _G
    epc = chunks * _K
    P = len(tables)
    C = tables[0].shape[1]
    D = 4
    U = 32
    rpt = n_acc // _NS
    assert n_acc % (_NS * 8) == 0

    zeros_c = jnp.zeros((rpt, C), jnp.float32)
    with_ea = ea is not None

    out_type = [jax.ShapeDtypeStruct((P * _NC, n_acc, C), jnp.float32)]
    scratch = [pltpu.VMEM((4, _G, _K), jnp.int32),
               pltpu.VMEM((4, _G, _K), jnp.int32)]
    scratch += [pltpu.VMEM((_K, C), jnp.float32) for _ in range(D)]
    scratch += [pltpu.SemaphoreType.DMA for _ in range(4)]
    scratch += [pltpu.SemaphoreType.DMA for _ in range(4)]
    scratch += [pltpu.SemaphoreType.DMA for _ in range(D)]
    scratch += [pltpu.VMEM_SHARED((n_acc, C), jnp.float32)]
    inputs = [*tables, srcp, dstp, zeros_c]
    if with_ea:
        zeros_e = jnp.zeros((rpt, 8), jnp.float32)
        inputs += [ea, zeros_e]
        out_type += [jax.ShapeDtypeStruct((_NC, n_acc, 8), jnp.float32)]
        scratch += [pltpu.VMEM_SHARED((n_acc, 8), jnp.float32),
                    pltpu.VMEM((4, _G, _K, 8), jnp.float32)]
        scratch += [pltpu.SemaphoreType.DMA for _ in range(4)]

    mesh = plsc.VectorSubcoreMesh(core_axis_name="c", subcore_axis_name="s")

    def body(*refs):
        it = iter(refs)
        table_hs = [next(it) for _ in range(P)]
        src_h = next(it)
        dst_h = next(it)
        zc_h = next(it)
        if with_ea:
            ea_h = next(it)
            ze_h = next(it)
        out_h = next(it)
        if with_ea:
            oute_h = next(it)
        sgrp = next(it)
        dgrp = next(it)
        rbufs = [next(it) for _ in range(D)]
        ssems = [next(it) for _ in range(4)]
        dsems = [next(it) for _ in range(4)]
        gsems = [next(it) for _ in range(D)]
        acc = next(it)
        if with_ea:
            acc_e = next(it)
            egrp = next(it)
            esems = [next(it) for _ in range(4)]

        c = lax.axis_index("c")
        s = lax.axis_index("s")
        wid = c * _NS + s
        r0 = s * rpt
        my_src = src_h.at[wid]
        my_dst = dst_h.at[wid]
        cbase = wid * chunks

        def src_load(g, slot):
            return pltpu.make_async_copy(my_src.at[g], sgrp.at[slot],
                                         ssems[slot])

        def dst_load(g, slot):
            return pltpu.make_async_copy(my_dst.at[g], dgrp.at[slot],
                                         dsems[slot])

        def ea_load(g, slot):
            return pltpu.make_async_copy(ea_h.at[pl.ds(cbase + g * _G, _G)],
                                         egrp.at[slot], esems[slot])

        def gather(table_h, kslot, u, b4):
            return pltpu.make_async_copy(table_h.at[sgrp.at[kslot].at[u]],
                                         rbufs[b4], gsems[b4])

        def step(o, u32, table_h, ea_now):
            j = o * U + u32
            k = u32 // 8
            u = u32 % 8
            b4 = u32 % 4
            g = o * 4 + k
            cpos = (u32 - D) % U
            kc = cpos // 8
            uc = cpos % 8
            jc = j - D

            if u == 0:
                @pl.when(g < ngroups)
                def _():
                    src_load(g, k).wait()
                    dst_load(g, k).wait()
                    if ea_now:
                        ea_load(g, k).wait()

                    @pl.when(g + 2 < ngroups)
                    def _():
                        src_load(g + 2, (k + 2) % 4).start()

                    @pl.when(g + 1 < ngroups)
                    def _():
                        dst_load(g + 1, (k + 1) % 4).start()
                        if ea_now:
                            ea_load(g + 1, (k + 1) % 4).start()

            @pl.when((jc >= 0) & (jc < chunks))
            def _():
                gather(table_h, kc, uc, b4).wait()
                didx = dgrp.at[kc].at[uc]
                pltpu.sync_copy(rbufs[b4], acc.at[didx], add=True)
                if ea_now:
                    pltpu.sync_copy(egrp.at[kc].at[uc],
                                    acc_e.at[didx], add=True)

            @pl.when(j < chunks)
            def _():
                gather(table_h, k, u, b4).start()

        nsteps = (chunks + D + U - 1) // U
        for p in range(P):
            table_h = table_hs[p]
            ea_now = with_ea and p == 0

            pltpu.sync_copy(zc_h, acc.at[pl.ds(r0, rpt)])
            if ea_now:
                pltpu.sync_copy(ze_h, acc_e.at[pl.ds(r0, rpt)])
            plsc.subcore_barrier()

            src_load(0, 0).start()
            src_load(1, 1).start()
            dst_load(0, 0).start()
            if ea_now:
                ea_load(0, 0).start()

            def outer(o, carry, table_h=table_h, ea_now=ea_now):
                for u32 in range(U):
                    step(o, u32, table_h, ea_now)
                return carry

            lax.fori_loop(0, nsteps, outer, 0)
            plsc.subcore_barrier()
            pltpu.sync_copy(acc.at[pl.ds(r0, rpt)],
                            out_h.at[p * _NC + c].at[pl.ds(r0, rpt)])
            if ea_now:
                pltpu.sync_copy(acc_e.at[pl.ds(r0, rpt)],
                                oute_h.at[c].at[pl.ds(r0, rpt)])

    f = pl.kernel(body, out_type=tuple(out_type), mesh=mesh,
                  scratch_types=scratch,
                  compiler_params=pltpu.CompilerParams(
                      use_tc_tiling_on_sc=False))
    res = f(*inputs)
    return res if with_ea else res[0]


_R = 2000


def _layer_tc(o, ae, hs, Whs, Wrs, We8, br, gamma, beta):
    B = o.shape[0] // 2
    Cb = o.shape[2]
    N = hs[0].shape[0]
    Co = Whs[0].shape[1]
    nb = N // _R
    inv_n = 1.0 / N
    nh = len(hs)

    def body(*refs):
        it = iter(refs)
        a_refs = [next(it) for _ in range(2 * B)]
        ae_refs = [next(it) for _ in range(2)]
        h_refs = [next(it) for _ in range(nh)]
        wh_refs = [next(it) for _ in range(B)]
        wr_refs = [next(it) for _ in range(nh)]
        we_r = next(it)
        br_r = next(it)
        g_r = next(it)
        b_r = next(it)
        h_out = next(it)
        ubuf = next(it)
        acc = next(it)

        i = pl.program_id(0)

        @pl.when(i == 0)
        def _():
            acc[...] = jnp.zeros_like(acc)

        @pl.when(i < nb)
        def _():
            u = (jnp.dot(ae_refs[0][0] + ae_refs[1][0], we_r[...],
                         preferred_element_type=jnp.float32, precision=_PREC)
                 + br_r[...])
            for b in range(B):
                u += jnp.dot(a_refs[2 * b][0] + a_refs[2 * b + 1][0],
                             wh_refs[b][...],
                             preferred_element_type=jnp.float32,
                             precision=_PREC)
            for kk in range(nh):
                u += jnp.dot(h_refs[kk][...], wr_refs[kk][...],
                             preferred_element_type=jnp.float32,
                             precision=_PREC)
            ubuf[pl.ds(jnp.minimum(i, nb - 1) * _R, _R), :] = u
            acc[0:1, :] += jnp.sum(u, axis=0, keepdims=True)
            acc[1:2, :] += jnp.sum(u * u, axis=0, keepdims=True)

        @pl.when(i >= nb)
        def _():
            mu = acc[0:1, :] * inv_n
            var = acc[1:2, :] * inv_n - mu * mu
            inv = lax.rsqrt(var + 1e-5)
            u = ubuf[pl.ds(jnp.maximum(i - nb, 0) * _R, _R), :]
            h_out[...] = jax.nn.gelu((u - mu) * inv * g_r[...] + b_r[...])

    p1 = lambda i: jnp.minimum(i, nb - 1)
    p2 = lambda i: jnp.where(i < nb, 0, i - nb)
    full = lambda a: pl.BlockSpec(a.shape, lambda i: (0,) * a.ndim)

    def plane_spec(pb):
        return pl.BlockSpec((1, _R, Cb), lambda i, pb=pb: (pb, p1(i), 0))

    def ae_spec(c_):
        return pl.BlockSpec((1, _R, 8),
                            lambda i, c_=c_: (c_, p1(i), 0))

    in_specs = ([plane_spec(pb) for pb in range(2 * B)]
                + [ae_spec(0), ae_spec(1)]
                + [pl.BlockSpec((_R, h.shape[1]),
                                lambda i, _w=h.shape[1]: (p1(i), 0))
                   for h in hs]
                + [full(w) for w in Whs] + [full(w) for w in Wrs]
                + [full(We8), full(br), full(gamma), full(beta)])
    h_out = pl.pallas_call(
        body,
        grid=(2 * nb,),
        in_specs=in_specs,
        out_specs=pl.BlockSpec((_R, Co), lambda i: (p2(i), 0)),
        out_shape=jax.ShapeDtypeStruct((N, Co), jnp.float32),
        scratch_shapes=[pltpu.VMEM((N, Co), jnp.float32),
                        pltpu.VMEM((8, Co), jnp.float32)],
    )(*([o] * (2 * B)), ae, ae, *hs, *Whs, *Wrs, We8, br, gamma, beta)
    return h_out


def _heads_tc(h4, batch3, zw, pw, emw, elw, ng):
    N = h4.shape[0]
    nb = N // _R
    zd = zw["W2a"].shape[1]
    eta = emw["W2"].shape[1]

    def body(hr, br_, W1, b1, W2a, b2a, W2b, b2b, Wsa, Wsb,
             Wp, bp, mW1, mb1, mW2, mb2, mWs, lW1, lb1, lW2, lb2, lWs,
             zmu_o, zsg_o, emu_o, esg_o, pool_acc, cnt_acc):
        i = pl.program_id(0)

        @pl.when(i == 0)
        def _():
            pool_acc[...] = jnp.zeros_like(pool_acc)
            cnt_acc[...] = jnp.zeros_like(cnt_acc)

        hb = hr[...]
        hh = jax.nn.gelu(jnp.dot(hb, W1[...],
                                 preferred_element_type=jnp.float32,
                                 precision=_PREC) + b1[...])
        zmu_o[...] = (jnp.dot(hh, W2a[...],
                              preferred_element_type=jnp.float32,
                              precision=_PREC)
                      + b2a[...]
                      + jnp.dot(hb, Wsa[...],
                                preferred_element_type=jnp.float32,
                                precision=_PREC))
        zs = (jnp.dot(hh, W2b[...], preferred_element_type=jnp.float32,
                      precision=_PREC)
              + b2b[...]
              + jnp.dot(hb, Wsb[...], preferred_element_type=jnp.float32,
                        precision=_PREC))
        zsg_o[...] = jnp.exp(jnp.clip(zs, -30.0, 20.0))

        bt = br_[0]
        oh = (bt == lax.broadcasted_iota(jnp.int32, (ng, 1), 0)
              ).astype(jnp.float32)
        pool_acc[...] += jnp.dot(oh, hb, preferred_element_type=jnp.float32,
                                 precision=_PREC)
        cnt_acc[...] += jnp.dot(oh, jnp.ones((_R, 128), jnp.float32),
                                preferred_element_type=jnp.float32,
                                precision=_PREC)

        @pl.when(i == nb - 1)
        def _():
            pooled = pool_acc[...] / jnp.maximum(cnt_acc[...], 1.0)
            g = jnp.dot(pooled, Wp[...], preferred_element_type=jnp.float32,
                        precision=_PREC) + bp[...]

            def rb(w1, bb1, w2, bb2, ws):
                t = jax.nn.gelu(jnp.dot(g, w1[...],
                                        preferred_element_type=jnp.float32,
                                        precision=_PREC) + bb1[...])
                return (jnp.dot(t, w2[...],
                                preferred_element_type=jnp.float32,
                                precision=_PREC)
                        + bb2[...]
                        + jnp.dot(g, ws[...],
                                  preferred_element_type=jnp.float32,
                                  precision=_PREC))

            emu_o[...] = rb(mW1, mb1, mW2, mb2, mWs)
            esg_o[...] = jnp.exp(jnp.clip(rb(lW1, lb1, lW2, lb2, lWs),
                                          -30.0, 20.0))

    row = lambda w: pl.BlockSpec((_R, w), lambda i: (i, 0))
    full = lambda a: pl.BlockSpec(a.shape, lambda i: (0,) * a.ndim)
    const = lambda shp: pl.BlockSpec(shp, lambda i: (0,) * len(shp))
    weights = [zw["W1"], zw["b1"], zw["W2a"], zw["b2a"], zw["W2b"], zw["b2b"],
               zw["Wsa"], zw["Wsb"], pw["W"], pw["b"],
               emw["W1"], emw["b1"], emw["W2"], emw["b2"], emw["Ws"],
               elw["W1"], elw["b1"], elw["W2"], elw["b2"], elw["Ws"]]
    return pl.pallas_call(
        body,
        grid=(nb,),
        in_specs=[row(128), pl.BlockSpec((1, 1, _R), lambda i: (i, 0, 0))]
                 + [full(w) for w in weights],
        out_specs=[row(zd), row(zd), const((ng, eta)), const((ng, eta))],
        out_shape=[jax.ShapeDtypeStruct((N, zd), jnp.float32),
                   jax.ShapeDtypeStruct((N, zd), jnp.float32),
                   jax.ShapeDtypeStruct((ng, eta), jnp.float32),
                   jax.ShapeDtypeStruct((ng, eta), jnp.float32)],
        scratch_shapes=[pltpu.VMEM((ng, 128), jnp.float32),
                        pltpu.VMEM((ng, 128), jnp.float32)],
    )(h4, batch3, *weights)


def kernel(x, edge_index, edge_attr, batch, params):
    N, _ = x.shape
    E = edge_index.shape[1]
    ng = 256
    src = edge_index[0].astype(jnp.int32)
    dst = edge_index[1].astype(jnp.int32)

    ngroups = -(-E // (_NW * _K * _G))
    chunks = _G * ngroups
    epad = _NW * _K * chunks - E
    srcp = jnp.concatenate([src, jnp.zeros((epad,), jnp.int32)]
                           ).reshape(_NW, ngroups, _G, _K)
    dstp = jnp.concatenate([dst, jnp.full((epad,), N, jnp.int32)]
                           ).reshape(_NW, ngroups, _G, _K)
    eap = jnp.pad(edge_attr.astype(jnp.float32).reshape(E // _K, _K, 4),
                  ((0, _NW * chunks - E // _K), (0, 0), (0, 4)),
                  constant_values=1.0)
    n_acc = (_NS * 8) * (-(-(N + 1) // (_NS * 8)))

    ax, ae = _sc_segsum([x], srcp, dstp, n_acc, ea=eap)

    h = x
    for i in range(4):
        p = params["conv%d" % i]
        ci = h.shape[1]
        if i == 0:
            o = ax
        elif ci <= 32:
            o = _sc_segsum([h], srcp, dstp, n_acc)
        else:
            o = _sc_segsum([h[:, :32], h[:, 32:]], srcp, dstp, n_acc)
        B = o.shape[0] // 2
        Cb = o.shape[2]
        Whs = [p["W_msg"][b * Cb:(b + 1) * Cb] for b in range(B)]
        co = p["W_msg"].shape[1]
        We8 = jnp.concatenate([p["W_msg"][ci:ci + 4], p["b_msg"][None, :],
                               jnp.zeros((3, co), jnp.float32)], axis=0)
        h = _layer_tc(o, ae, [h], Whs, [p["W_root"]], We8,
                      p["b_root"][None, :], p["gamma"][None, :],
                      p["beta"][None, :])

    zp = params["z_head"]
    zd = zp["W2"].shape[1] // 2
    zw = {"W1": zp["W1"], "b1": zp["b1"][None, :],
          "W2a": zp["W2"][:, :zd], "b2a": zp["b2"][None, :zd],
          "W2b": zp["W2"][:, zd:], "b2b": zp["b2"][None, zd:],
          "Wsa": zp["Wskip"][:, :zd], "Wsb": zp["Wskip"][:, zd:]}
    pw = {"W": params["pool"]["W"], "b": params["pool"]["b"][None, :]}

    def rbw(p):
        return {"W1": p["W1"], "b1": p["b1"][None, :], "W2": p["W2"],
                "b2": p["b2"][None, :], "Ws": p["Wskip"]}

    batch3 = batch.astype(jnp.int32).reshape(N // _R, 1, _R)
    z_mu, z_sigma, eta_mu, eta_sigma = _heads_tc(
        h, batch3, zw, pw, rbw(params["eta_mu"]), rbw(params["eta_ls"]), ng)
    return (z_mu, z_sigma, eta_mu, eta_sigma)


# speedup vs baseline: 1.1668x; 1.1668x over previous
import jax
import jax.numpy as jnp
from jax import lax
from jax.experimental import pallas as pl
from jax.experimental.pallas import tpu as pltpu
from jax.experimental.pallas import tpu_sc as plsc

_NC = 2
_NS = 16
_NW = _NC * _NS
_K = 128
_G = 8
_PREC = lax.Precision.HIGHEST


def _sc_segsum(tables, srcp, dstp, n_acc, ea=None):
    ngroups = srcp.shape[1]
    chunks = ngroups * _G
    epc = chunks * _K
    P = len(tables)
    C = tables[0].shape[1]
    D = 4
    U = 32
    rpt = n_acc // _NS
    assert n_acc % (_NS * 8) == 0

    zeros_c = jnp.zeros((rpt, C), jnp.float32)
    with_ea = ea is not None

    out_type = [jax.ShapeDtypeStruct((P * _NC, n_acc, C), jnp.float32)]
    scratch = [pltpu.VMEM((4, _G, _K), jnp.int32),
               pltpu.VMEM((4, _G, _K), jnp.int32)]
    scratch += [pltpu.VMEM((_K, C), jnp.float32) for _ in range(D)]
    scratch += [pltpu.SemaphoreType.DMA for _ in range(4)]
    scratch += [pltpu.SemaphoreType.DMA for _ in range(4)]
    scratch += [pltpu.SemaphoreType.DMA for _ in range(D)]
    scratch += [pltpu.VMEM_SHARED((n_acc, C), jnp.float32)]
    inputs = [*tables, srcp, dstp, zeros_c]
    if with_ea:
        zeros_e = jnp.zeros((rpt, 8), jnp.float32)
        inputs += [ea, zeros_e]
        out_type += [jax.ShapeDtypeStruct((_NC, n_acc, 8), jnp.float32)]
        scratch += [pltpu.VMEM_SHARED((n_acc, 8), jnp.float32),
                    pltpu.VMEM((4, _G, _K, 8), jnp.float32)]
        scratch += [pltpu.SemaphoreType.DMA for _ in range(4)]

    mesh = plsc.VectorSubcoreMesh(core_axis_name="c", subcore_axis_name="s")

    def body(*refs):
        it = iter(refs)
        table_hs = [next(it) for _ in range(P)]
        src_h = next(it)
        dst_h = next(it)
        zc_h = next(it)
        if with_ea:
            ea_h = next(it)
            ze_h = next(it)
        out_h = next(it)
        if with_ea:
            oute_h = next(it)
        sgrp = next(it)
        dgrp = next(it)
        rbufs = [next(it) for _ in range(D)]
        ssems = [next(it) for _ in range(4)]
        dsems = [next(it) for _ in range(4)]
        gsems = [next(it) for _ in range(D)]
        acc = next(it)
        if with_ea:
            acc_e = next(it)
            egrp = next(it)
            esems = [next(it) for _ in range(4)]

        c = lax.axis_index("c")
        s = lax.axis_index("s")
        wid = c * _NS + s
        r0 = s * rpt
        my_src = src_h.at[wid]
        my_dst = dst_h.at[wid]

        def src_load(g, slot):
            return pltpu.make_async_copy(my_src.at[g], sgrp.at[slot],
                                         ssems[slot])

        def dst_load(g, slot):
            return pltpu.make_async_copy(my_dst.at[g], dgrp.at[slot],
                                         dsems[slot])

        def ea_load(g, slot):
            return pltpu.make_async_copy(ea_h.at[wid].at[pl.ds(g * _G, _G)],
                                         egrp.at[slot], esems[slot])

        def gather(table_h, kslot, u, b4):
            return pltpu.make_async_copy(table_h.at[sgrp.at[kslot].at[u]],
                                         rbufs[b4], gsems[b4])

        def step(o, u32, table_h, ea_now):
            j = o * U + u32
            k = u32 // 8
            u = u32 % 8
            b4 = u32 % 4
            g = o * 4 + k
            cpos = (u32 - D) % U
            kc = cpos // 8
            uc = cpos % 8
            jc = j - D

            if u == 0:
                @pl.when(g < ngroups)
                def _():
                    src_load(g, k).wait()
                    dst_load(g, k).wait()
                    if ea_now:
                        ea_load(g, k).wait()

                    @pl.when(g + 2 < ngroups)
                    def _():
                        src_load(g + 2, (k + 2) % 4).start()

                    @pl.when(g + 1 < ngroups)
                    def _():
                        dst_load(g + 1, (k + 1) % 4).start()
                        if ea_now:
                            ea_load(g + 1, (k + 1) % 4).start()

            @pl.when((jc >= 0) & (jc < chunks))
            def _():
                gather(table_h, kc, uc, b4).wait()
                didx = dgrp.at[kc].at[uc]
                pltpu.sync_copy(rbufs[b4], acc.at[didx], add=True)
                if ea_now:
                    pltpu.sync_copy(egrp.at[kc].at[uc],
                                    acc_e.at[didx], add=True)

            @pl.when(j < chunks)
            def _():
                gather(table_h, k, u, b4).start()

        nsteps = (chunks + D + U - 1) // U
        for p in range(P):
            table_h = table_hs[p]
            ea_now = with_ea and p == 0

            pltpu.sync_copy(zc_h, acc.at[pl.ds(r0, rpt)])
            if ea_now:
                pltpu.sync_copy(ze_h, acc_e.at[pl.ds(r0, rpt)])
            plsc.subcore_barrier()

            src_load(0, 0).start()
            src_load(1, 1).start()
            dst_load(0, 0).start()
            if ea_now:
                ea_load(0, 0).start()

            def outer(o, carry, table_h=table_h, ea_now=ea_now):
                for u32 in range(U):
                    step(o, u32, table_h, ea_now)
                return carry

            lax.fori_loop(0, nsteps, outer, 0)
            plsc.subcore_barrier()
            pltpu.sync_copy(acc.at[pl.ds(r0, rpt)],
                            out_h.at[p * _NC + c].at[pl.ds(r0, rpt)])
            if ea_now:
                pltpu.sync_copy(acc_e.at[pl.ds(r0, rpt)],
                                oute_h.at[c].at[pl.ds(r0, rpt)])

    f = pl.kernel(body, out_type=tuple(out_type), mesh=mesh,
                  scratch_types=scratch,
                  compiler_params=pltpu.CompilerParams(
                      use_tc_tiling_on_sc=False))
    res = f(*inputs)
    return res if with_ea else res[0]


_R = 2000


def _layer_tc(o, ae, hs, Whs, Wrs, We8, br, gamma, beta):
    B = o.shape[0] // 2
    Cb = o.shape[2]
    N = hs[0].shape[0]
    Co = Whs[0].shape[1]
    nb = N // _R
    inv_n = 1.0 / N
    nh = len(hs)

    def body(*refs):
        it = iter(refs)
        a_refs = [next(it) for _ in range(2 * B)]
        ae_refs = [next(it) for _ in range(2)]
        h_refs = [next(it) for _ in range(nh)]
        wh_refs = [next(it) for _ in range(B)]
        wr_refs = [next(it) for _ in range(nh)]
        we_r = next(it)
        br_r = next(it)
        g_r = next(it)
        b_r = next(it)
        h_out = next(it)
        ubuf = next(it)
        acc = next(it)

        i = pl.program_id(0)

        @pl.when(i == 0)
        def _():
            acc[...] = jnp.zeros_like(acc)

        @pl.when(i < nb)
        def _():
            u = (jnp.dot(ae_refs[0][0] + ae_refs[1][0], we_r[...],
                         preferred_element_type=jnp.float32, precision=_PREC)
                 + br_r[...])
            for b in range(B):
                u += jnp.dot(a_refs[2 * b][0] + a_refs[2 * b + 1][0],
                             wh_refs[b][...],
                             preferred_element_type=jnp.float32,
                             precision=_PREC)
            for kk in range(nh):
                u += jnp.dot(h_refs[kk][...], wr_refs[kk][...],
                             preferred_element_type=jnp.float32,
                             precision=_PREC)
            ubuf[pl.ds(jnp.minimum(i, nb - 1) * _R, _R), :] = u
            acc[0:1, :] += jnp.sum(u, axis=0, keepdims=True)
            acc[1:2, :] += jnp.sum(u * u, axis=0, keepdims=True)

        @pl.when(i >= nb)
        def _():
            mu = acc[0:1, :] * inv_n
            var = acc[1:2, :] * inv_n - mu * mu
            inv = lax.rsqrt(var + 1e-5)
            u = ubuf[pl.ds(jnp.maximum(i - nb, 0) * _R, _R), :]
            h_out[...] = jax.nn.gelu((u - mu) * inv * g_r[...] + b_r[...])

    p1 = lambda i: jnp.minimum(i, nb - 1)
    p2 = lambda i: jnp.where(i < nb, 0, i - nb)
    full = lambda a: pl.BlockSpec(a.shape, lambda i: (0,) * a.ndim)

    def plane_spec(pb):
        return pl.BlockSpec((1, _R, Cb), lambda i, pb=pb: (pb, p1(i), 0))

    def ae_spec(c_):
        return pl.BlockSpec((1, _R, 8),
                            lambda i, c_=c_: (c_, p1(i), 0))

    in_specs = ([plane_spec(pb) for pb in range(2 * B)]
                + [ae_spec(0), ae_spec(1)]
                + [pl.BlockSpec((_R, h.shape[1]),
                                lambda i, _w=h.shape[1]: (p1(i), 0))
                   for h in hs]
                + [full(w) for w in Whs] + [full(w) for w in Wrs]
                + [full(We8), full(br), full(gamma), full(beta)])
    h_out = pl.pallas_call(
        body,
        grid=(2 * nb,),
        in_specs=in_specs,
        out_specs=pl.BlockSpec((_R, Co), lambda i: (p2(i), 0)),
        out_shape=jax.ShapeDtypeStruct((N, Co), jnp.float32),
        scratch_shapes=[pltpu.VMEM((N, Co), jnp.float32),
                        pltpu.VMEM((8, Co), jnp.float32)],
    )(*([o] * (2 * B)), ae, ae, *hs, *Whs, *Wrs, We8, br, gamma, beta)
    return h_out


def _heads_tc(h4, batch3, zw, pw, emw, elw, ng):
    N = h4.shape[0]
    nb = N // _R
    zd = zw["W2a"].shape[1]
    eta = emw["W2"].shape[1]

    def body(hr, br_, W1, b1, W2a, b2a, W2b, b2b, Wsa, Wsb,
             Wp, bp, mW1, mb1, mW2, mb2, mWs, lW1, lb1, lW2, lb2, lWs,
             zmu_o, zsg_o, emu_o, esg_o, pool_acc, cnt_acc):
        i = pl.program_id(0)

        @pl.when(i == 0)
        def _():
            pool_acc[...] = jnp.zeros_like(pool_acc)
            cnt_acc[...] = jnp.zeros_like(cnt_acc)

        hb = hr[...]
        hh = jax.nn.gelu(jnp.dot(hb, W1[...],
                                 preferred_element_type=jnp.float32,
                                 precision=_PREC) + b1[...])
        zmu_o[...] = (jnp.dot(hh, W2a[...],
                              preferred_element_type=jnp.float32,
                              precision=_PREC)
                      + b2a[...]
                      + jnp.dot(hb, Wsa[...],
                                preferred_element_type=jnp.float32,
                                precision=_PREC))
        zs = (jnp.dot(hh, W2b[...], preferred_element_type=jnp.float32,
                      precision=_PREC)
              + b2b[...]
              + jnp.dot(hb, Wsb[...], preferred_element_type=jnp.float32,
                        precision=_PREC))
        zsg_o[...] = jnp.exp(jnp.clip(zs, -30.0, 20.0))

        bt = br_[0]
        oh = (bt == lax.broadcasted_iota(jnp.int32, (ng, 1), 0)
              ).astype(jnp.float32)
        pool_acc[...] += jnp.dot(oh, hb, preferred_element_type=jnp.float32,
                                 precision=_PREC)
        cnt_acc[...] += jnp.dot(oh, jnp.ones((_R, 128), jnp.float32),
                                preferred_element_type=jnp.float32,
                                precision=_PREC)

        @pl.when(i == nb - 1)
        def _():
            pooled = pool_acc[...] / jnp.maximum(cnt_acc[...], 1.0)
            g = jnp.dot(pooled, Wp[...], preferred_element_type=jnp.float32,
                        precision=_PREC) + bp[...]

            def rb(w1, bb1, w2, bb2, ws):
                t = jax.nn.gelu(jnp.dot(g, w1[...],
                                        preferred_element_type=jnp.float32,
                                        precision=_PREC) + bb1[...])
                return (jnp.dot(t, w2[...],
                                preferred_element_type=jnp.float32,
                                precision=_PREC)
                        + bb2[...]
                        + jnp.dot(g, ws[...],
                                  preferred_element_type=jnp.float32,
                                  precision=_PREC))

            emu_o[...] = rb(mW1, mb1, mW2, mb2, mWs)
            esg_o[...] = jnp.exp(jnp.clip(rb(lW1, lb1, lW2, lb2, lWs),
                                          -30.0, 20.0))

    row = lambda w: pl.BlockSpec((_R, w), lambda i: (i, 0))
    full = lambda a: pl.BlockSpec(a.shape, lambda i: (0,) * a.ndim)
    const = lambda shp: pl.BlockSpec(shp, lambda i: (0,) * len(shp))
    weights = [zw["W1"], zw["b1"], zw["W2a"], zw["b2a"], zw["W2b"], zw["b2b"],
               zw["Wsa"], zw["Wsb"], pw["W"], pw["b"],
               emw["W1"], emw["b1"], emw["W2"], emw["b2"], emw["Ws"],
               elw["W1"], elw["b1"], elw["W2"], elw["b2"], elw["Ws"]]
    return pl.pallas_call(
        body,
        grid=(nb,),
        in_specs=[row(128), pl.BlockSpec((1, 1, _R), lambda i: (i, 0, 0))]
                 + [full(w) for w in weights],
        out_specs=[row(zd), row(zd), const((ng, eta)), const((ng, eta))],
        out_shape=[jax.ShapeDtypeStruct((N, zd), jnp.float32),
                   jax.ShapeDtypeStruct((N, zd), jnp.float32),
                   jax.ShapeDtypeStruct((ng, eta), jnp.float32),
                   jax.ShapeDtypeStruct((ng, eta), jnp.float32)],
        scratch_shapes=[pltpu.VMEM((ng, 128), jnp.float32),
                        pltpu.VMEM((ng, 128), jnp.float32)],
    )(h4, batch3, *weights)


def _ea8_tc(ea, chunks):
    E = ea.shape[0]
    rows_b = 6400
    cpb = rows_b // _K
    nvalid = E // rows_b
    ntot = _NW * chunks // cpb

    def body(e_r, o_r):
        i = pl.program_id(0)
        v = jnp.concatenate([e_r[...], jnp.ones((rows_b, 4), jnp.float32)],
                            axis=1)
        o_r[...] = jnp.where(i < nvalid, v, 1.0).reshape(1, cpb, _K, 8)

    return pl.pallas_call(
        body,
        grid=(ntot,),
        in_specs=[pl.BlockSpec((rows_b, 4),
                               lambda i: (jnp.minimum(i, nvalid - 1), 0))],
        out_specs=pl.BlockSpec((1, cpb, _K, 8),
                               lambda i: (i // (chunks // cpb),
                                          i % (chunks // cpb), 0, 0)),
        out_shape=jax.ShapeDtypeStruct((_NW, chunks, _K, 8), jnp.float32),
    )(ea)


def kernel(x, edge_index, edge_attr, batch, params):
    N, _ = x.shape
    E = edge_index.shape[1]
    ng = 256
    src = edge_index[0].astype(jnp.int32)
    dst = edge_index[1].astype(jnp.int32)

    ngroups = -(-E // (_NW * _K * _G))
    chunks = _G * ngroups
    epad = _NW * _K * chunks - E
    srcp = jnp.concatenate([src, jnp.zeros((epad,), jnp.int32)]
                           ).reshape(_NW, ngroups, _G, _K)
    dstp = jnp.concatenate([dst, jnp.full((epad,), N, jnp.int32)]
                           ).reshape(_NW, ngroups, _G, _K)
    eap = _ea8_tc(edge_attr.astype(jnp.float32), chunks)
    n_acc = (_NS * 8) * (-(-(N + 1) // (_NS * 8)))

    ax, ae = _sc_segsum([x], srcp, dstp, n_acc, ea=eap)

    h = x
    for i in range(4):
        p = params["conv%d" % i]
        ci = h.shape[1]
        if i == 0:
            o = ax
        elif ci <= 32:
            o = _sc_segsum([h], srcp, dstp, n_acc)
        else:
            o = _sc_segsum([h[:, :32], h[:, 32:]], srcp, dstp, n_acc)
        B = o.shape[0] // 2
        Cb = o.shape[2]
        Whs = [p["W_msg"][b * Cb:(b + 1) * Cb] for b in range(B)]
        co = p["W_msg"].shape[1]
        We8 = jnp.concatenate([p["W_msg"][ci:ci + 4], p["b_msg"][None, :],
                               jnp.zeros((3, co), jnp.float32)], axis=0)
        h = _layer_tc(o, ae, [h], Whs, [p["W_root"]], We8,
                      p["b_root"][None, :], p["gamma"][None, :],
                      p["beta"][None, :])

    zp = params["z_head"]
    zd = zp["W2"].shape[1] // 2
    zw = {"W1": zp["W1"], "b1": zp["b1"][None, :],
          "W2a": zp["W2"][:, :zd], "b2a": zp["b2"][None, :zd],
          "W2b": zp["W2"][:, zd:], "b2b": zp["b2"][None, zd:],
          "Wsa": zp["Wskip"][:, :zd], "Wsb": zp["Wskip"][:, zd:]}
    pw = {"W": params["pool"]["W"], "b": params["pool"]["b"][None, :]}

    def rbw(p):
        return {"W1": p["W1"], "b1": p["b1"][None, :], "W2": p["W2"],
                "b2": p["b2"][None, :], "Ws": p["Wskip"]}

    batch3 = batch.astype(jnp.int32).reshape(N // _R, 1, _R)
    z_mu, z_sigma, eta_mu, eta_sigma = _heads_tc(
        h, batch3, zw, pw, rbw(params["eta_mu"]), rbw(params["eta_ls"]), ng)
    return (z_mu, z_sigma, eta_mu, eta_sigma)
